# Initial kernel scaffold; baseline (speedup 1.0000x reference)
import jax
import jax.numpy as jnp
from jax.experimental import pallas as pl


def kernel(input_x, embedding, W1, b1, W2, b2):
    raise NotImplementedError("write your pallas kernel here")


# speedup vs baseline: 14.5861x; 14.5861x over previous
import functools

import jax
import jax.numpy as jnp
from jax import lax
from jax.experimental import pallas as pl
from jax.experimental.pallas import tpu as pltpu
from jax.experimental.pallas import tpu_sc as plsc

_VOCAB = 1000000
_EMB = 32
_BATCH = 16384
_SEQ = 200
_FC = 128
_NCLS = 10

_NC = 2
_NS = 16
_NW = _NC * _NS
_B_PER_W = _BATCH // _NW
_HALF = _SEQ // 2
_CH = 256
_NCHUNK = _B_PER_W // _CH
_L = 16


def _pool_body(idx_hbm, table_hbm, out_hbm, idx_v, rows_v, out_v, sem0, sem1):
    wid = lax.axis_index("s") * _NC + lax.axis_index("c")
    base = wid * _B_PER_W
    sems = (sem0, sem1)
    inv_seq = 1.0 / _SEQ

    def fire(local_r, buf):
        pltpu.async_copy(
            table_hbm.at[idx_v.at[2 * local_r]],
            rows_v.at[buf, pl.ds(0, _HALF)], sems[buf])
        pltpu.async_copy(
            table_hbm.at[idx_v.at[2 * local_r + 1]],
            rows_v.at[buf, pl.ds(_HALF, _HALF)], sems[buf])

    def wait(buf):
        pltpu.make_async_copy(
            table_hbm.at[idx_v.at[0]],
            rows_v.at[buf, pl.ds(0, _HALF)], sems[buf]).wait()
        pltpu.make_async_copy(
            table_hbm.at[idx_v.at[0]],
            rows_v.at[buf, pl.ds(_HALF, _HALF)], sems[buf]).wait()

    def accumulate(buf, out_slot):
        zero = jnp.zeros((_L,), jnp.float32)

        def acc_j(j, accs):
            a0, a1 = accs
            a0 = a0 + rows_v[buf, j, pl.ds(0, _L)]
            a1 = a1 + rows_v[buf, j, pl.ds(_L, _L)]
            return (a0, a1)

        a0, a1 = lax.fori_loop(0, _SEQ, acc_j, (zero, zero), unroll=8)
        out_v[out_slot, pl.ds(0, _L)] = a0 * inv_seq
        out_v[out_slot, pl.ds(_L, _L)] = a1 * inv_seq

    for c in range(_NCHUNK):
        row0 = base + c * _CH
        pltpu.sync_copy(idx_hbm.at[pl.ds(2 * row0, 2 * _CH)], idx_v)
        fire(0, 0)
        fire(1, 1)

        def pair(p, _):
            r0 = 2 * p
            wait(0)

            @pl.when(r0 + 2 < _CH)
            def _():
                fire(r0 + 2, 0)

            accumulate(0, c * _CH + r0)
            wait(1)

            @pl.when(r0 + 3 < _CH)
            def _():
                fire(r0 + 3, 1)

            accumulate(1, c * _CH + r0 + 1)
            return 0

        lax.fori_loop(0, _CH // 2, pair, 0)

    pltpu.sync_copy(out_v, out_hbm.at[pl.ds(base, _B_PER_W)])


@functools.partial(jax.jit, static_argnames=())
def _pool(input_x, embedding):
    idx2 = input_x.reshape(2 * _BATCH, _HALF)
    mesh = plsc.VectorSubcoreMesh(
        core_axis_name="c", subcore_axis_name="s",
        num_cores=_NC, num_subcores=_NS)
    return pl.kernel(
        _pool_body,
        out_type=jax.ShapeDtypeStruct((_BATCH, _EMB), jnp.float32),
        mesh=mesh,
        compiler_params=pltpu.CompilerParams(use_tc_tiling_on_sc=False),
        scratch_types=[
            pltpu.VMEM((2 * _CH, _HALF), jnp.int32),
            pltpu.VMEM((2, _SEQ, _EMB), jnp.float32),
            pltpu.VMEM((_B_PER_W, _EMB), jnp.float32),
            pltpu.SemaphoreType.DMA,
            pltpu.SemaphoreType.DMA,
        ],
    )(idx2, embedding)


def _fc_body(x_ref, w1_ref, b1_ref, w2_ref, b2_ref, o_ref):
    x = x_ref[...]
    h = jnp.maximum(
        jnp.dot(x, w1_ref[...], preferred_element_type=jnp.float32)
        + b1_ref[...], 0.0)
    o_ref[...] = (
        jnp.dot(h, w2_ref[...], preferred_element_type=jnp.float32)
        + b2_ref[...])


def _fc(pooled, W1, b1, W2, b2):
    bm = 2048
    grid = (_BATCH // bm,)
    return pl.pallas_call(
        _fc_body,
        grid=grid,
        in_specs=[
            pl.BlockSpec((bm, _EMB), lambda i: (i, 0)),
            pl.BlockSpec((_EMB, _FC), lambda i: (0, 0)),
            pl.BlockSpec((1, _FC), lambda i: (0, 0)),
            pl.BlockSpec((_FC, _NCLS), lambda i: (0, 0)),
            pl.BlockSpec((1, _NCLS), lambda i: (0, 0)),
        ],
        out_specs=pl.BlockSpec((bm, _NCLS), lambda i: (i, 0)),
        out_shape=jax.ShapeDtypeStruct((_BATCH, _NCLS), jnp.float32),
    )(pooled, W1, b1.reshape(1, _FC), W2, b2.reshape(1, _NCLS))


def kernel(input_x, embedding, W1, b1, W2, b2):
    pooled = _pool(input_x, embedding)
    return _fc(pooled, W1, b1, W2, b2)


# speedup vs baseline: 15.6502x; 1.0730x over previous
import functools

import jax
import jax.numpy as jnp
from jax import lax
from jax.experimental import pallas as pl
from jax.experimental.pallas import tpu as pltpu
from jax.experimental.pallas import tpu_sc as plsc

_VOCAB = 1000000
_EMB = 32
_BATCH = 16384
_SEQ = 200
_FC = 128
_NCLS = 10

_NC = 2
_NS = 16
_NW = _NC * _NS
_B_PER_W = _BATCH // _NW
_SPLIT0 = 96
_SPLIT1 = _SEQ - _SPLIT0
_CH = 256
_NCHUNK = _B_PER_W // _CH
_L = 16

_PACK_COLS = 2048
_PACK_ROWS = _PACK_COLS // 4
_PACK_GRID = -(-_VOCAB // _PACK_COLS)
_TAIL_I0 = (_PACK_GRID - 1) * _PACK_COLS
_TAIL_N = _VOCAB - _TAIL_I0
_TAIL_Q = _TAIL_N // 4


def _pack_body(x_ref, o_ref):
    t = jnp.transpose(x_ref[...])
    g = pl.program_id(0)

    @pl.when(g < _PACK_GRID - 1)
    def _():
        for k in range(4):
            o_ref[:, 32 * k:32 * (k + 1)] = \
                t[_PACK_ROWS * k:_PACK_ROWS * (k + 1), :]

    @pl.when(g == _PACK_GRID - 1)
    def _():
        for k in range(4):
            o_ref[0:_TAIL_Q, 32 * k:32 * (k + 1)] = \
                t[_TAIL_Q * k:_TAIL_Q * (k + 1), :]


def _pack(embT):
    return pl.pallas_call(
        _pack_body,
        grid=(_PACK_GRID,),
        in_specs=[pl.BlockSpec((_EMB, _PACK_COLS), lambda i: (0, i))],
        out_specs=pl.BlockSpec((_PACK_ROWS, 128), lambda i: (i, 0)),
        out_shape=jax.ShapeDtypeStruct((_VOCAB // 4, 128), jnp.float32),
    )(embT)


def _pool_body(idx_hbm, table_hbm, out_hbm, idx_v, rows_v, out_v, sem0, sem1):
    wid = lax.axis_index("s") * _NC + lax.axis_index("c")
    base = wid * _B_PER_W
    sems = (sem0, sem1)
    inv_seq = 1.0 / _SEQ

    def fire(local_r, buf):
        pltpu.async_copy(
            table_hbm.at[idx_v.at[local_r, pl.ds(0, _SPLIT0)]],
            rows_v.at[buf, pl.ds(0, _SPLIT0)], sems[buf])
        pltpu.async_copy(
            table_hbm.at[idx_v.at[local_r, pl.ds(_SPLIT0, _SPLIT1)]],
            rows_v.at[buf, pl.ds(_SPLIT0, _SPLIT1)], sems[buf])

    def wait(buf):
        pltpu.make_async_copy(
            table_hbm.at[idx_v.at[0, pl.ds(0, _SPLIT0)]],
            rows_v.at[buf, pl.ds(0, _SPLIT0)], sems[buf]).wait()
        pltpu.make_async_copy(
            table_hbm.at[idx_v.at[0, pl.ds(_SPLIT0, _SPLIT1)]],
            rows_v.at[buf, pl.ds(_SPLIT0, _SPLIT1)], sems[buf]).wait()

    def accumulate(buf, out_slot):
        zero = jnp.zeros((_L,), jnp.float32)

        def acc_j(j, accs):
            a0, a1 = accs
            a0 = a0 + rows_v[buf, j, pl.ds(0, _L)]
            a1 = a1 + rows_v[buf, j, pl.ds(_L, _L)]
            return (a0, a1)

        a0, a1 = lax.fori_loop(0, _SEQ, acc_j, (zero, zero), unroll=8)
        out_v[out_slot, pl.ds(0, _L)] = a0 * inv_seq
        out_v[out_slot, pl.ds(_L, _L)] = a1 * inv_seq

    for c in range(_NCHUNK):
        row0 = base + c * _CH
        pltpu.sync_copy(idx_hbm.at[pl.ds(row0, _CH)], idx_v)
        fire(0, 0)
        fire(1, 1)

        def pair(p, _):
            r0 = 2 * p
            wait(0)

            @pl.when(r0 + 2 < _CH)
            def _():
                fire(r0 + 2, 0)

            accumulate(0, c * _CH + r0)
            wait(1)

            @pl.when(r0 + 3 < _CH)
            def _():
                fire(r0 + 3, 1)

            accumulate(1, c * _CH + r0 + 1)
            return 0

        lax.fori_loop(0, _CH // 2, pair, 0)

    pltpu.sync_copy(out_v, out_hbm.at[pl.ds(base, _B_PER_W)])


def _pool(pidx, table):
    mesh = plsc.VectorSubcoreMesh(
        core_axis_name="c", subcore_axis_name="s",
        num_cores=_NC, num_subcores=_NS)
    return pl.kernel(
        _pool_body,
        out_type=jax.ShapeDtypeStruct((_BATCH, _EMB), jnp.float32),
        mesh=mesh,
        compiler_params=pltpu.CompilerParams(use_tc_tiling_on_sc=False),
        scratch_types=[
            pltpu.VMEM((_CH, _SEQ), jnp.int32),
            pltpu.VMEM((2, _SEQ, _EMB), jnp.float32),
            pltpu.VMEM((_B_PER_W, _EMB), jnp.float32),
            pltpu.SemaphoreType.DMA,
            pltpu.SemaphoreType.DMA,
        ],
    )(pidx, table)


def _fc_body(x_ref, w1_ref, b1_ref, w2_ref, b2_ref, o_ref):
    x = x_ref[...]
    h = jnp.maximum(
        jnp.dot(x, w1_ref[...], preferred_element_type=jnp.float32)
        + b1_ref[...], 0.0)
    o_ref[...] = (
        jnp.dot(h, w2_ref[...], preferred_element_type=jnp.float32)
        + b2_ref[...])


def _fc(pooled, W1, b1, W2, b2):
    bm = 2048
    return pl.pallas_call(
        _fc_body,
        grid=(_BATCH // bm,),
        in_specs=[
            pl.BlockSpec((bm, _EMB), lambda i: (i, 0)),
            pl.BlockSpec((_EMB, _FC), lambda i: (0, 0)),
            pl.BlockSpec((1, _FC), lambda i: (0, 0)),
            pl.BlockSpec((_FC, _NCLS), lambda i: (0, 0)),
            pl.BlockSpec((1, _NCLS), lambda i: (0, 0)),
        ],
        out_specs=pl.BlockSpec((bm, _NCLS), lambda i: (i, 0)),
        out_shape=jax.ShapeDtypeStruct((_BATCH, _NCLS), jnp.float32),
    )(pooled, W1, b1.reshape(1, _FC), W2, b2.reshape(1, _NCLS))


def kernel(input_x, embedding, W1, b1, W2, b2):
    p_main = ((input_x & ~2047)
              | ((input_x & 511) << 2)
              | ((input_x & 2047) >> 9))
    r_tail = input_x - _TAIL_I0
    p_tail = _TAIL_I0 + (r_tail % _TAIL_Q) * 4 + r_tail // _TAIL_Q
    pidx = jnp.where(input_x < _TAIL_I0, p_main, p_tail)
    packed = _pack(embedding.T)
    table = packed.reshape(_VOCAB, _EMB)
    pooled = _pool(pidx, table)
    return _fc(pooled, W1, b1, W2, b2)
